# Initial kernel scaffold; baseline (speedup 1.0000x reference)
import functools

import jax
import jax.numpy as jnp
from jax import lax
from jax.experimental import pallas as pl
from jax.experimental.pallas import tpu as pltpu
from jax.experimental.pallas import tpu_sc as plsc

NC = 2
NS = 16
NW = NC * NS
LANES = 16
EDGE_CHUNK = 80


def _make_deg(E, NPAD):
    EW = E // NW
    mesh = plsc.VectorSubcoreMesh(core_axis_name="c", subcore_axis_name="s")

    @functools.partial(
        pl.kernel,
        out_type=jax.ShapeDtypeStruct((NW, NPAD), jnp.float32),
        mesh=mesh,
        scratch_types=[
            pltpu.VMEM((EW,), jnp.int32),
            pltpu.VMEM((NPAD,), jnp.float32),
        ],
    )
    def deg_kernel(dst_hbm, out_hbm, idx_v, cnt_v):
        w = lax.axis_index("c") * NS + lax.axis_index("s")
        zero16 = jnp.zeros((LANES,), jnp.float32)
        one16 = jnp.ones((LANES,), jnp.float32)

        def zero_body(i, carry):
            cnt_v[pl.ds(i * LANES, LANES)] = zero16
            return carry

        lax.fori_loop(0, NPAD // LANES, zero_body, 0)
        pltpu.sync_copy(dst_hbm.at[pl.ds(w * EW, EW)], idx_v)

        def acc_body(i, carry):
            idx = idx_v[pl.ds(i * LANES, LANES)]
            plsc.addupdate_scatter(cnt_v, [idx], one16)
            return carry

        lax.fori_loop(0, EW // LANES, acc_body, 0)
        pltpu.sync_copy(cnt_v, out_hbm.at[w])

    return deg_kernel


def _make_agg(N, D, E, dst_shared):
    nchunks = E // ((NS if dst_shared else NW) * EDGE_CHUNK)
    rows_t = N // NS
    mesh = plsc.VectorSubcoreMesh(core_axis_name="c", subcore_axis_name="s")

    @functools.partial(
        pl.kernel,
        out_type=jax.ShapeDtypeStruct((NC * N, D), jnp.float32),
        mesh=mesh,
        scratch_types=[
            pltpu.VMEM((nchunks, EDGE_CHUNK), jnp.int32),
            pltpu.VMEM((nchunks, EDGE_CHUNK), jnp.int32),
            pltpu.VMEM((EDGE_CHUNK, D), jnp.float32),
            pltpu.VMEM_SHARED((N, D), jnp.float32),
            pltpu.SemaphoreType.DMA,
        ],
    )
    def agg_kernel(tbl_hbm, src_hbm, dst_hbm, zero_hbm, out_hbm,
                   src_v, dst_v, rows_v, acc, sem):
        c = lax.axis_index("c")
        s = lax.axis_index("s")
        w = c * NS + s
        pltpu.sync_copy(zero_hbm.at[pl.ds(s * rows_t, rows_t)],
                        acc.at[pl.ds(s * rows_t, rows_t)])
        pltpu.sync_copy(src_hbm.at[pl.ds(w * nchunks, nchunks)], src_v)
        dbase = (s if dst_shared else w) * nchunks
        pltpu.sync_copy(dst_hbm.at[pl.ds(dbase, nchunks)], dst_v)
        plsc.subcore_barrier()

        def step(i, carry):
            pltpu.async_copy(tbl_hbm.at[src_v.at[i]], rows_v, sem).wait()
            pltpu.sync_copy(rows_v, acc.at[dst_v.at[i]], add=True)
            return carry

        lax.fori_loop(0, nchunks, step, 0)
        plsc.subcore_barrier()
        pltpu.sync_copy(acc.at[pl.ds(s * rows_t, rows_t)],
                        out_hbm.at[pl.ds(c * N + s * rows_t, rows_t)])

    return agg_kernel


def _ab_body(x_ref, w1_ref, degp_ref, u1_ref, dinv_ref):
    deg = jnp.sum(degp_ref[...], axis=0) + 1.0
    dinv = lax.rsqrt(deg)
    h = jnp.dot(x_ref[...], w1_ref[...], preferred_element_type=jnp.float32)
    u1_ref[0] = h * dinv[:, None]
    dinv_ref[0] = dinv


def _mid_body(agg_ref, u1_ref, dinv_ref, b1_ref, g1_ref, be1_ref, w2_ref,
              u2_ref, *, hd, d2):
    dinv = dinv_ref[0][:, None]
    r0 = jnp.maximum((agg_ref[0] + u1_ref[0]) * dinv + b1_ref[0, :hd], 0.0)
    r1 = jnp.maximum((agg_ref[1] + u1_ref[1]) * dinv + b1_ref[0, hd:], 0.0)
    m = (jnp.sum(r0, -1, keepdims=True) + jnp.sum(r1, -1, keepdims=True)) / d2
    v = (jnp.sum((r0 - m) ** 2, -1, keepdims=True)
         + jnp.sum((r1 - m) ** 2, -1, keepdims=True)) / d2
    inv = lax.rsqrt(v + 1e-5)
    n0 = (r0 - m) * inv * g1_ref[0, :hd] + be1_ref[0, :hd]
    n1 = (r1 - m) * inv * g1_ref[0, hd:] + be1_ref[0, hd:]
    h2 = (jnp.dot(n0, w2_ref[:hd], preferred_element_type=jnp.float32)
          + jnp.dot(n1, w2_ref[hd:], preferred_element_type=jnp.float32))
    u2_ref[...] = h2 * dinv


def _fin_body(p_ref, u2_ref, dinv_ref, b2_ref, o_ref):
    o_ref[...] = ((p_ref[0] + p_ref[1] + u2_ref[...])
                  * dinv_ref[0][:, None] + b2_ref[0])


def kernel(x_list, edge_index_list, W1, b1, gamma1, beta1, W2, b2):
    Gn, N, D1 = x_list.shape
    D2 = W1.shape[1]
    D3 = W2.shape[1]
    HD = D2 // 2
    E = edge_index_list.shape[2]
    NPAD = -(-N // 256) * 256
    R = 1000
    NBLK = N // R

    deg_call = _make_deg(E, NPAD)
    agg1_call = _make_agg(N, HD, E, dst_shared=True)
    agg2_call = _make_agg(N, D3, E, dst_shared=False)

    ab_call = pl.pallas_call(
        _ab_body,
        grid=(NBLK, 2),
        in_specs=[
            pl.BlockSpec((R, D1), lambda i, h: (i, 0)),
            pl.BlockSpec((D1, HD), lambda i, h: (0, h)),
            pl.BlockSpec((NW, R), lambda i, h: (0, i)),
        ],
        out_specs=[
            pl.BlockSpec((1, R, HD), lambda i, h: (h, i, 0)),
            pl.BlockSpec((1, R), lambda i, h: (0, i)),
        ],
        out_shape=[
            jax.ShapeDtypeStruct((2, N, HD), jnp.float32),
            jax.ShapeDtypeStruct((1, N), jnp.float32),
        ],
    )
    mid_call = pl.pallas_call(
        functools.partial(_mid_body, hd=HD, d2=D2),
        grid=(NBLK,),
        in_specs=[
            pl.BlockSpec((2, R, HD), lambda i: (0, i, 0)),
            pl.BlockSpec((2, R, HD), lambda i: (0, i, 0)),
            pl.BlockSpec((1, R), lambda i: (0, i)),
            pl.BlockSpec((1, D2), lambda i: (0, 0)),
            pl.BlockSpec((1, D2), lambda i: (0, 0)),
            pl.BlockSpec((1, D2), lambda i: (0, 0)),
            pl.BlockSpec((D2, D3), lambda i: (0, 0)),
        ],
        out_specs=pl.BlockSpec((R, D3), lambda i: (i, 0)),
        out_shape=jax.ShapeDtypeStruct((N, D3), jnp.float32),
    )
    fin_call = pl.pallas_call(
        _fin_body,
        grid=(NBLK,),
        in_specs=[
            pl.BlockSpec((2, R, D3), lambda i: (0, i, 0)),
            pl.BlockSpec((R, D3), lambda i: (i, 0)),
            pl.BlockSpec((1, R), lambda i: (0, i)),
            pl.BlockSpec((1, D3), lambda i: (0, 0)),
        ],
        out_specs=pl.BlockSpec((R, D3), lambda i: (i, 0)),
        out_shape=jax.ShapeDtypeStruct((N, D3), jnp.float32),
    )

    zeros_tbl = jnp.zeros((N, HD), jnp.float32)
    b1r = b1.reshape(1, D2)
    g1r = gamma1.reshape(1, D2)
    be1r = beta1.reshape(1, D2)
    b2r = b2.reshape(1, D3)

    outs = []
    for g in range(Gn):
        x = x_list[g]
        src = edge_index_list[g, 0]
        dst = edge_index_list[g, 1]
        dst2 = dst.reshape(E // EDGE_CHUNK, EDGE_CHUNK)
        src_l1 = jnp.concatenate([src, src + N]).reshape(
            2 * E // EDGE_CHUNK, EDGE_CHUNK)
        src_l2 = src.reshape(E // EDGE_CHUNK, EDGE_CHUNK)

        degp = deg_call(dst)
        u1, dinv = ab_call(x, W1, degp)
        agg1 = agg1_call(u1.reshape(2 * N, HD), src_l1, dst2,
                         zeros_tbl).reshape(2, N, HD)
        u2 = mid_call(agg1, u1, dinv, b1r, g1r, be1r, W2)
        p = agg2_call(u2, src_l2, dst2, zeros_tbl).reshape(2, N, D3)
        outs.append(fin_call(p, u2, dinv, b2r))
    return jnp.stack(outs)


# speedup vs baseline: 5.4887x; 5.4887x over previous
import functools

import jax
import jax.numpy as jnp
from jax import lax
from jax.experimental import pallas as pl
from jax.experimental.pallas import tpu as pltpu
from jax.experimental.pallas import tpu_sc as plsc

NC = 2
NS = 16
NW = NC * NS
LANES = 16
HD = 128
EDGE_CHUNK = 128
IBLK = 20


def _make_deg(EPAD, NPAD):
    nchunks = EPAD // (NW * EDGE_CHUNK)
    rows_t = NPAD // NS
    mesh = plsc.VectorSubcoreMesh(core_axis_name="c", subcore_axis_name="s")

    @functools.partial(
        pl.kernel,
        out_type=jax.ShapeDtypeStruct((NC * NPAD, LANES), jnp.float32),
        mesh=mesh,
        scratch_types=[
            pltpu.VMEM((nchunks, EDGE_CHUNK), jnp.int32),
            pltpu.VMEM((EDGE_CHUNK, LANES), jnp.float32),
            pltpu.VMEM_SHARED((NPAD, LANES), jnp.float32),
        ],
    )
    def deg_kernel(dst_hbm, ones_hbm, zero_hbm, out_hbm, dst_v, ones_v, acc):
        c = lax.axis_index("c")
        s = lax.axis_index("s")
        w = c * NS + s
        pltpu.sync_copy(zero_hbm.at[pl.ds(s * rows_t, rows_t)],
                        acc.at[pl.ds(s * rows_t, rows_t)])
        pltpu.sync_copy(dst_hbm.at[w], dst_v)
        pltpu.sync_copy(ones_hbm, ones_v)
        plsc.subcore_barrier()

        def step(i, carry):
            pltpu.sync_copy(ones_v, acc.at[dst_v.at[i]], add=True)
            return carry

        lax.fori_loop(0, nchunks, step, 0)
        plsc.subcore_barrier()
        pltpu.sync_copy(acc.at[pl.ds(s * rows_t, rows_t)],
                        out_hbm.at[pl.ds(c * NPAD + s * rows_t, rows_t)])

    return deg_kernel


def _make_agg(NPAD, EPAD):
    nchunks = EPAD // (NS * EDGE_CHUNK)
    nblk = nchunks // IBLK
    rows_t = NPAD // NS
    mesh = plsc.VectorSubcoreMesh(core_axis_name="c", subcore_axis_name="s")

    @functools.partial(
        pl.kernel,
        out_type=jax.ShapeDtypeStruct((NC * NPAD, HD), jnp.float32),
        mesh=mesh,
        scratch_types=[
            pltpu.VMEM((IBLK, EDGE_CHUNK), jnp.int32),
            pltpu.VMEM((IBLK, EDGE_CHUNK), jnp.int32),
            pltpu.VMEM((EDGE_CHUNK, HD), jnp.float32),
            pltpu.VMEM_SHARED((NPAD, HD), jnp.float32),
            pltpu.SemaphoreType.DMA,
        ],
    )
    def agg_kernel(tbl_hbm, src_hbm, dst_hbm, zero_hbm, out_hbm,
                   src_v, dst_v, rows_v, acc, sem):
        c = lax.axis_index("c")
        s = lax.axis_index("s")
        pltpu.sync_copy(zero_hbm.at[pl.ds(s * rows_t, rows_t)],
                        acc.at[pl.ds(s * rows_t, rows_t)])
        plsc.subcore_barrier()

        def outer(j, carry):
            pltpu.sync_copy(dst_hbm.at[s].at[j], dst_v)
            pltpu.sync_copy(src_hbm.at[c * NS + s].at[j], src_v)

            def step(i, carry2):
                pltpu.async_copy(tbl_hbm.at[src_v.at[i]], rows_v, sem).wait()
                pltpu.sync_copy(rows_v, acc.at[dst_v.at[i]], add=True)
                return carry2

            lax.fori_loop(0, IBLK, step, 0)
            return carry

        lax.fori_loop(0, nblk, outer, 0)
        plsc.subcore_barrier()
        pltpu.sync_copy(acc.at[pl.ds(s * rows_t, rows_t)],
                        out_hbm.at[pl.ds(c * NPAD + s * rows_t, rows_t)])

    return agg_kernel


def _dk_body(degp_ref, dinv_ref, *, npad):
    deg = (degp_ref[0:npad, 0:1] + degp_ref[npad:, 0:1]) + 1.0
    dinv_ref[...] = lax.rsqrt(deg)


def _ab_body(x_ref, w1_ref, dinv_ref, u1_ref):
    h = jnp.dot(x_ref[...], w1_ref[...], preferred_element_type=jnp.float32)
    dinv = dinv_ref[...]
    u1_ref[0] = h[:, :HD] * dinv
    u1_ref[1] = h[:, HD:] * dinv


def _mid_body(agg_ref, u1_ref, dinv_ref, b1_ref, g1_ref, be1_ref, w2_ref,
              u2_ref, *, d2):
    dinv = dinv_ref[...]
    r0 = jnp.maximum((agg_ref[0] + u1_ref[0]) * dinv + b1_ref[0, :HD], 0.0)
    r1 = jnp.maximum((agg_ref[1] + u1_ref[1]) * dinv + b1_ref[0, HD:], 0.0)
    m = (jnp.sum(r0, -1, keepdims=True) + jnp.sum(r1, -1, keepdims=True)) / d2
    v = (jnp.sum((r0 - m) ** 2, -1, keepdims=True)
         + jnp.sum((r1 - m) ** 2, -1, keepdims=True)) / d2
    inv = lax.rsqrt(v + 1e-5)
    n0 = (r0 - m) * inv * g1_ref[0, :HD] + be1_ref[0, :HD]
    n1 = (r1 - m) * inv * g1_ref[0, HD:] + be1_ref[0, HD:]
    h2 = (jnp.dot(n0, w2_ref[:HD], preferred_element_type=jnp.float32)
          + jnp.dot(n1, w2_ref[HD:], preferred_element_type=jnp.float32))
    u2 = h2 * dinv
    u2_ref[0] = u2
    u2_ref[1] = u2


def _fin_body(p_ref, u2_ref, dinv_ref, b2_ref, o_ref):
    o_ref[...] = (p_ref[0] + u2_ref[0]) * dinv_ref[...] + b2_ref[0]


def kernel(x_list, edge_index_list, W1, b1, gamma1, beta1, W2, b2):
    Gn, N, D1 = x_list.shape
    D2 = W1.shape[1]
    D3 = W2.shape[1]
    E = edge_index_list.shape[2]
    NPAD = -(-N // 256) * 256
    egran = NS * EDGE_CHUNK * IBLK * NC
    EPAD = -(-E // egran) * egran
    R = 1000
    NBLK = N // R

    deg_call = _make_deg(EPAD, NPAD)
    agg_call = _make_agg(NPAD, EPAD)

    dk_call = pl.pallas_call(
        functools.partial(_dk_body, npad=NPAD),
        grid=(1,),
        in_specs=[pl.BlockSpec((NC * NPAD, LANES), lambda i: (0, 0))],
        out_specs=pl.BlockSpec((NPAD, 1), lambda i: (0, 0)),
        out_shape=jax.ShapeDtypeStruct((NPAD, 1), jnp.float32),
    )
    ab_call = pl.pallas_call(
        _ab_body,
        grid=(NBLK,),
        in_specs=[
            pl.BlockSpec((R, D1), lambda i: (i, 0)),
            pl.BlockSpec((D1, D2), lambda i: (0, 0)),
            pl.BlockSpec((R, 1), lambda i: (i, 0)),
        ],
        out_specs=pl.BlockSpec((2, R, HD), lambda i: (0, i, 0)),
        out_shape=jax.ShapeDtypeStruct((2, N, HD), jnp.float32),
    )
    mid_call = pl.pallas_call(
        functools.partial(_mid_body, d2=D2),
        grid=(NBLK,),
        in_specs=[
            pl.BlockSpec((2, R, HD), lambda i: (0, i, 0)),
            pl.BlockSpec((2, R, HD), lambda i: (0, i, 0)),
            pl.BlockSpec((R, 1), lambda i: (i, 0)),
            pl.BlockSpec((1, D2), lambda i: (0, 0)),
            pl.BlockSpec((1, D2), lambda i: (0, 0)),
            pl.BlockSpec((1, D2), lambda i: (0, 0)),
            pl.BlockSpec((D2, D3), lambda i: (0, 0)),
        ],
        out_specs=pl.BlockSpec((2, R, D3), lambda i: (0, i, 0)),
        out_shape=jax.ShapeDtypeStruct((2, N, D3), jnp.float32),
    )
    fin_call = pl.pallas_call(
        _fin_body,
        grid=(NBLK,),
        in_specs=[
            pl.BlockSpec((1, R, D3), lambda i: (0, i, 0)),
            pl.BlockSpec((1, R, D3), lambda i: (0, i, 0)),
            pl.BlockSpec((R, 1), lambda i: (i, 0)),
            pl.BlockSpec((1, D3), lambda i: (0, 0)),
        ],
        out_specs=pl.BlockSpec((R, D3), lambda i: (i, 0)),
        out_shape=jax.ShapeDtypeStruct((N, D3), jnp.float32),
    )

    zeros_tbl = jnp.zeros((NPAD, HD), jnp.float32)
    ones16 = jnp.ones((EDGE_CHUNK, LANES), jnp.float32)
    zeros16 = jnp.zeros((NPAD, LANES), jnp.float32)
    b1r = b1.reshape(1, D2)
    g1r = gamma1.reshape(1, D2)
    be1r = beta1.reshape(1, D2)
    b2r = b2.reshape(1, D3)
    npadlen = EPAD - E

    outs = []
    for g in range(Gn):
        x = x_list[g]
        src = edge_index_list[g, 0]
        dst = edge_index_list[g, 1]
        src_p = jnp.concatenate([src, jnp.zeros((npadlen,), jnp.int32)])
        dst_p = jnp.concatenate([dst, jnp.full((npadlen,), N, jnp.int32)])
        nblk = EPAD // (NS * EDGE_CHUNK * IBLK)
        dst_s = dst_p.reshape(NS, nblk, IBLK, EDGE_CHUNK)
        dst_w = dst_p.reshape(NW, EPAD // (NW * EDGE_CHUNK), EDGE_CHUNK)
        src_l = jnp.concatenate([src_p, src_p + N]).reshape(
            2 * NS, nblk, IBLK, EDGE_CHUNK)

        degp = deg_call(dst_w, ones16, zeros16)
        dinv = dk_call(degp)
        u1 = ab_call(x, W1, dinv)
        agg1 = agg_call(u1.reshape(2 * N, HD), src_l, dst_s,
                        zeros_tbl).reshape(2, NPAD, HD)
        u2 = mid_call(agg1, u1, dinv, b1r, g1r, be1r, W2)
        p = agg_call(u2.reshape(2 * N, D3), src_l, dst_s,
                     zeros_tbl).reshape(2, NPAD, D3)
        outs.append(fin_call(p, u2, dinv, b2r))
    return jnp.stack(outs)
